# baseline (speedup 1.0000x reference)
import functools

import jax
import jax.numpy as jnp
from jax import lax
from jax.experimental import pallas as pl
from jax.experimental.pallas import tpu as pltpu
from jax.experimental.pallas import tpu_sc as plsc

B = 16384
D = 64
DP = 128
NROWS = 1000000
PROWS = NROWS // 2
NC = 2
NS = 16
NW = NC * NS
RC = 320
PC = RC // 2
RCHUNKS = NROWS // RC
RN = -(-RCHUNKS // NW)
BPW = B // NW
ICH = 128
GCH = 256
NGC = BPW // GCH

_mesh = plsc.VectorSubcoreMesh(core_axis_name="c", subcore_axis_name="s")
_params = pltpu.CompilerParams(
    needs_layout_passes=False, use_tc_tiling_on_sc=True)


@functools.partial(
    pl.kernel,
    mesh=_mesh,
    out_type=(jax.ShapeDtypeStruct((PROWS, DP), jnp.float32),
              jax.ShapeDtypeStruct((PROWS, DP), jnp.float32)),
    scratch_types=[
        pltpu.VMEM((2, RC, D), jnp.float32),
        pltpu.VMEM((2, PC, DP), jnp.float32),
        pltpu.SemaphoreType.DMA,
        pltpu.SemaphoreType.DMA,
        pltpu.SemaphoreType.DMA,
        pltpu.SemaphoreType.DMA,
    ],
    compiler_params=_params,
)
def _detile(utab_hbm, itab_hbm, uscr_hbm, iscr_hbm,
            raw_v, pk_v, semr0, semr1, semw0, semw1):
    wid = lax.axis_index("s") * NC + lax.axis_index("c")

    def rd(tab, t, slot, sem):
        return pltpu.make_async_copy(
            tab.at[pl.ds((t * NW + wid) * RC, RC)], raw_v.at[slot], sem)

    def wr(scr, t, slot, sem):
        return pltpu.make_async_copy(
            pk_v.at[slot], scr.at[pl.ds((t * NW + wid) * PC, PC)], sem)

    def merge(slot):
        def mbody(k, carry):
            for q in range(D // 16):
                pk_v[slot, k, pl.ds(q * 16, 16)] = (
                    raw_v[slot, 2 * k, pl.ds(q * 16, 16)])
                pk_v[slot, k, pl.ds(D + q * 16, 16)] = (
                    raw_v[slot, 2 * k + 1, pl.ds(q * 16, 16)])
            return carry
        lax.fori_loop(0, PC, mbody, 0)

    def pipeline(tab, scr):
        sems = (semr0, semr1)
        semw = (semw0, semw1)
        rd(tab, 0, 0, semr0).start()

        def body(g, carry):
            for s in range(2):
                t = 2 * g + s
                nxt = t + 1

                @pl.when(t * NW + wid < RCHUNKS)
                def _():
                    rd(tab, t, s, sems[s]).wait()

                    @pl.when(nxt * NW + wid < RCHUNKS)
                    def _():
                        rd(tab, nxt, 1 - s, sems[1 - s]).start()

                    @pl.when(g > 0)
                    def _():
                        wr(scr, t, s, semw[s]).wait()

                    merge(s)
                    wr(scr, t, s, semw[s]).start()

            return carry

        lax.fori_loop(0, RN // 2, body, 0)
        wr(scr, 0, 0, semw0).wait()
        wr(scr, 0, 1, semw1).wait()

    pipeline(utab_hbm, uscr_hbm)
    pipeline(itab_hbm, iscr_hbm)


@functools.partial(
    pl.kernel,
    mesh=_mesh,
    out_type=jax.ShapeDtypeStruct((B,), jnp.float32),
    scratch_types=[
        pltpu.VMEM((BPW,), jnp.int32),
        pltpu.VMEM((BPW // ICH, ICH), jnp.int32),
        pltpu.VMEM((BPW // ICH, ICH), jnp.int32),
        pltpu.VMEM((BPW,), jnp.int32),
        pltpu.VMEM((BPW,), jnp.int32),
        pltpu.VMEM((GCH, DP), jnp.float32),
        pltpu.VMEM((GCH, DP), jnp.float32),
        pltpu.VMEM((BPW,), jnp.float32),
        pltpu.SemaphoreType.DMA,
    ],
    compiler_params=_params,
)
def _gather_dot(uidx_hbm, iidx_hbm, uscr_hbm, iscr_hbm, out_hbm,
                raw_v, uhx_v, ihx_v, uoff_v, ioff_v,
                urows_v, irows_v, out_v, sem):
    wid = lax.axis_index("s") * NC + lax.axis_index("c")
    base = wid * BPW

    for idx_hbm, hx_v, off_v in ((uidx_hbm, uhx_v, uoff_v),
                                 (iidx_hbm, ihx_v, ioff_v)):
        pltpu.sync_copy(idx_hbm.at[pl.ds(base, BPW)], raw_v)

        def sbody(k, carry, hx_v=hx_v, off_v=off_v):
            def inner(kk):
                for jj in range(ICH // 16):
                    v = raw_v[pl.ds(kk * ICH + jj * 16, 16)]
                    hx_v[kk, pl.ds(jj * 16, 16)] = v >> 1
                    off_v[pl.ds(kk * ICH + jj * 16, 16)] = (v & 1) * D
            for kk in range(BPW // ICH):
                pl.when(k == kk)(lambda kk=kk: inner(kk))
            return carry

        lax.fori_loop(0, BPW // ICH, sbody, 0)

    iota16 = lax.iota(jnp.int32, 16)

    for g in range(NGC):
        copies = []
        for j in range(GCH // ICH):
            jj = g * (GCH // ICH) + j
            copies.append(pltpu.make_async_copy(
                uscr_hbm.at[uhx_v.at[jj]],
                urows_v.at[pl.ds(j * ICH, ICH)], sem))
            copies.append(pltpu.make_async_copy(
                iscr_hbm.at[ihx_v.at[jj]],
                irows_v.at[pl.ds(j * ICH, ICH)], sem))
        for cp in copies:
            cp.start()
        for cp in copies:
            cp.wait()

        def body(k, carry):
            lb = k * 16
            uov = uoff_v[pl.ds(g * GCH + lb, 16)]
            iov = ioff_v[pl.ds(g * GCH + lb, 16)]
            s = jnp.zeros((16,), jnp.float32)
            for j in range(16):
                r = lb + j
                uo = uov[j]
                io = iov[j]
                acc = (urows_v[r, pl.ds(uo, 16)]
                       * irows_v[r, pl.ds(io, 16)])
                for q in range(1, D // 16):
                    acc = acc + (urows_v[r, pl.ds(uo + q * 16, 16)]
                                 * irows_v[r, pl.ds(io + q * 16, 16)])
                tot = jnp.sum(acc)
                s = lax.select(iota16 == j, lax.broadcast(tot, (16,)), s)
            out_v[pl.ds(g * GCH + lb, 16)] = s
            return carry

        lax.fori_loop(0, GCH // 16, body, 0)

    pltpu.sync_copy(out_v, out_hbm.at[pl.ds(base, BPW)])


def kernel(user_input, item_input, user_table, item_table):
    uscr, iscr = _detile(user_table, item_table)
    out = _gather_dot(user_input, item_input, uscr, iscr)
    return out.reshape(B, 1)


# speedup vs baseline: 1.0205x; 1.0205x over previous
import functools

import jax
import jax.numpy as jnp
from jax import lax
from jax.experimental import pallas as pl
from jax.experimental.pallas import tpu as pltpu
from jax.experimental.pallas import tpu_sc as plsc

B = 16384
D = 64
DP = 128
NROWS = 1000000
PROWS = NROWS // 2
NC = 2
NS = 16
NW = NC * NS
RC = 320
PC = RC // 2
RCHUNKS = NROWS // RC
RN = -(-RCHUNKS // NW)
BPW = B // NW
ICH = 128
GCH = 256
NGC = BPW // GCH

_mesh = plsc.VectorSubcoreMesh(core_axis_name="c", subcore_axis_name="s")
_params = pltpu.CompilerParams(
    needs_layout_passes=False, use_tc_tiling_on_sc=True)


@functools.partial(
    pl.kernel,
    mesh=_mesh,
    out_type=(jax.ShapeDtypeStruct((PROWS, DP), jnp.float32),
              jax.ShapeDtypeStruct((PROWS, DP), jnp.float32)),
    scratch_types=[
        pltpu.VMEM((2, RC, D), jnp.float32),
        pltpu.VMEM((2, PC, DP), jnp.float32),
        pltpu.SemaphoreType.DMA,
        pltpu.SemaphoreType.DMA,
        pltpu.SemaphoreType.DMA,
        pltpu.SemaphoreType.DMA,
    ],
    compiler_params=_params,
)
def _detile(utab_hbm, itab_hbm, uscr_hbm, iscr_hbm,
            raw_v, pk_v, semr0, semr1, semw0, semw1):
    wid = lax.axis_index("s") * NC + lax.axis_index("c")

    def rd(tab, t, slot, sem):
        return pltpu.make_async_copy(
            tab.at[pl.ds((t * NW + wid) * RC, RC)], raw_v.at[slot], sem)

    def wr(scr, t, slot, sem):
        return pltpu.make_async_copy(
            pk_v.at[slot], scr.at[pl.ds((t * NW + wid) * PC, PC)], sem)

    def merge(slot):
        def mbody(k, carry):
            for q in range(D // 16):
                pk_v[slot, k, pl.ds(q * 16, 16)] = (
                    raw_v[slot, 2 * k, pl.ds(q * 16, 16)])
                pk_v[slot, k, pl.ds(D + q * 16, 16)] = (
                    raw_v[slot, 2 * k + 1, pl.ds(q * 16, 16)])
            return carry
        lax.fori_loop(0, PC, mbody, 0, unroll=8)

    def pipeline(tab, scr):
        sems = (semr0, semr1)
        semw = (semw0, semw1)
        rd(tab, 0, 0, semr0).start()

        def body(g, carry):
            for s in range(2):
                t = 2 * g + s
                nxt = t + 1

                @pl.when(t * NW + wid < RCHUNKS)
                def _():
                    rd(tab, t, s, sems[s]).wait()

                    @pl.when(nxt * NW + wid < RCHUNKS)
                    def _():
                        rd(tab, nxt, 1 - s, sems[1 - s]).start()

                    @pl.when(g > 0)
                    def _():
                        wr(scr, t, s, semw[s]).wait()

                    merge(s)
                    wr(scr, t, s, semw[s]).start()

            return carry

        lax.fori_loop(0, RN // 2, body, 0)
        wr(scr, 0, 0, semw0).wait()
        wr(scr, 0, 1, semw1).wait()

    pipeline(utab_hbm, uscr_hbm)
    pipeline(itab_hbm, iscr_hbm)


@functools.partial(
    pl.kernel,
    mesh=_mesh,
    out_type=jax.ShapeDtypeStruct((B,), jnp.float32),
    scratch_types=[
        pltpu.VMEM((BPW,), jnp.int32),
        pltpu.VMEM((BPW // ICH, ICH), jnp.int32),
        pltpu.VMEM((BPW // ICH, ICH), jnp.int32),
        pltpu.VMEM((BPW,), jnp.int32),
        pltpu.VMEM((BPW,), jnp.int32),
        pltpu.VMEM((GCH, DP), jnp.float32),
        pltpu.VMEM((GCH, DP), jnp.float32),
        pltpu.VMEM((BPW,), jnp.float32),
        pltpu.SemaphoreType.DMA,
    ],
    compiler_params=_params,
)
def _gather_dot(uidx_hbm, iidx_hbm, uscr_hbm, iscr_hbm, out_hbm,
                raw_v, uhx_v, ihx_v, uoff_v, ioff_v,
                urows_v, irows_v, out_v, sem):
    wid = lax.axis_index("s") * NC + lax.axis_index("c")
    base = wid * BPW

    for idx_hbm, hx_v, off_v in ((uidx_hbm, uhx_v, uoff_v),
                                 (iidx_hbm, ihx_v, ioff_v)):
        pltpu.sync_copy(idx_hbm.at[pl.ds(base, BPW)], raw_v)

        def sbody(k, carry, hx_v=hx_v, off_v=off_v):
            def inner(kk):
                for jj in range(ICH // 16):
                    v = raw_v[pl.ds(kk * ICH + jj * 16, 16)]
                    hx_v[kk, pl.ds(jj * 16, 16)] = v >> 1
                    off_v[pl.ds(kk * ICH + jj * 16, 16)] = (v & 1) * D
            for kk in range(BPW // ICH):
                pl.when(k == kk)(lambda kk=kk: inner(kk))
            return carry

        lax.fori_loop(0, BPW // ICH, sbody, 0)

    iota16 = lax.iota(jnp.int32, 16)

    for g in range(NGC):
        copies = []
        for j in range(GCH // ICH):
            jj = g * (GCH // ICH) + j
            copies.append(pltpu.make_async_copy(
                uscr_hbm.at[uhx_v.at[jj]],
                urows_v.at[pl.ds(j * ICH, ICH)], sem))
            copies.append(pltpu.make_async_copy(
                iscr_hbm.at[ihx_v.at[jj]],
                irows_v.at[pl.ds(j * ICH, ICH)], sem))
        for cp in copies:
            cp.start()
        for cp in copies:
            cp.wait()

        def body(k, carry):
            lb = k * 16
            uov = uoff_v[pl.ds(g * GCH + lb, 16)]
            iov = ioff_v[pl.ds(g * GCH + lb, 16)]
            s = jnp.zeros((16,), jnp.float32)
            for j in range(16):
                r = lb + j
                uo = uov[j]
                io = iov[j]
                acc = (urows_v[r, pl.ds(uo, 16)]
                       * irows_v[r, pl.ds(io, 16)])
                for q in range(1, D // 16):
                    acc = acc + (urows_v[r, pl.ds(uo + q * 16, 16)]
                                 * irows_v[r, pl.ds(io + q * 16, 16)])
                tot = jnp.sum(acc)
                s = lax.select(iota16 == j, lax.broadcast(tot, (16,)), s)
            out_v[pl.ds(g * GCH + lb, 16)] = s
            return carry

        lax.fori_loop(0, GCH // 16, body, 0)

    pltpu.sync_copy(out_v, out_hbm.at[pl.ds(base, BPW)])


def kernel(user_input, item_input, user_table, item_table):
    uscr, iscr = _detile(user_table, item_table)
    out = _gather_dot(user_input, item_input, uscr, iscr)
    return out.reshape(B, 1)


# speedup vs baseline: 2.2894x; 2.2435x over previous
import functools

import jax
import jax.numpy as jnp
from jax import lax
from jax.experimental import pallas as pl
from jax.experimental.pallas import tpu as pltpu
from jax.experimental.pallas import tpu_sc as plsc

B = 16384
D = 64
NC = 2
NS = 16
NW = NC * NS
BPW = B // NW
CH = 256
NCHK = BPW // CH

_mesh = plsc.VectorSubcoreMesh(core_axis_name="c", subcore_axis_name="s")


@functools.partial(
    pl.kernel,
    mesh=_mesh,
    out_type=jax.ShapeDtypeStruct((B,), jnp.float32),
    scratch_types=[
        pltpu.VMEM((BPW,), jnp.int32),
        pltpu.VMEM((BPW,), jnp.int32),
        pltpu.VMEM((CH, D), jnp.float32),
        pltpu.VMEM((CH, D), jnp.float32),
        pltpu.VMEM((BPW,), jnp.float32),
        pltpu.SemaphoreType.DMA,
    ],
    compiler_params=pltpu.CompilerParams(
        needs_layout_passes=False, use_tc_tiling_on_sc=True),
)
def _sc_dot(uidx_hbm, iidx_hbm, utab_hbm, itab_hbm, out_hbm,
            uidx_v, iidx_v, urows_v, irows_v, out_v, sem):
    wid = lax.axis_index("s") * NC + lax.axis_index("c")
    base = wid * BPW

    pltpu.sync_copy(uidx_hbm.at[pl.ds(base, BPW)], uidx_v)
    pltpu.sync_copy(iidx_hbm.at[pl.ds(base, BPW)], iidx_v)

    iota16 = lax.iota(jnp.int32, 16)

    def chunk(c, carry_c):
        cbase = c * CH

        def dma_body(g, carry):
            rb = cbase + g * 16
            uvec = uidx_v[pl.ds(rb, 16)]
            ivec = iidx_v[pl.ds(rb, 16)]
            lb = g * 16
            for j in range(16):
                iu = uvec[j]
                ii = ivec[j]
                pltpu.make_async_copy(
                    utab_hbm.at[pl.ds(iu, 1)],
                    urows_v.at[pl.ds(lb + j, 1)], sem).start()
                pltpu.make_async_copy(
                    itab_hbm.at[pl.ds(ii, 1)],
                    irows_v.at[pl.ds(lb + j, 1)], sem).start()
            return carry

        lax.fori_loop(0, CH // 16, dma_body, 0)

        pltpu.make_async_copy(
            utab_hbm.at[pl.ds(0, CH)], urows_v, sem).wait()
        pltpu.make_async_copy(
            itab_hbm.at[pl.ds(0, CH)], irows_v, sem).wait()

        def body(g, carry):
            lb = g * 16
            s = jnp.zeros((16,), jnp.float32)
            for j in range(16):
                r = lb + j
                acc = urows_v[r, pl.ds(0, 16)] * irows_v[r, pl.ds(0, 16)]
                for q in range(1, D // 16):
                    acc = acc + (urows_v[r, pl.ds(q * 16, 16)]
                                 * irows_v[r, pl.ds(q * 16, 16)])
                tot = jnp.sum(acc)
                s = lax.select(iota16 == j, lax.broadcast(tot, (16,)), s)
            out_v[pl.ds(cbase + lb, 16)] = s
            return carry

        lax.fori_loop(0, CH // 16, body, 0)
        return carry_c

    lax.fori_loop(0, NCHK, chunk, 0)

    pltpu.sync_copy(out_v, out_hbm.at[pl.ds(base, BPW)])


def kernel(user_input, item_input, user_table, item_table):
    out = _sc_dot(user_input, item_input, user_table, item_table)
    return out.reshape(B, 1)
